# Initial kernel scaffold; baseline (speedup 1.0000x reference)
import jax
import jax.numpy as jnp
from jax.experimental import pallas as pl


def kernel(boxes, scores):
    raise NotImplementedError("write your pallas kernel here")


# speedup vs baseline: 1.2709x; 1.2709x over previous
import jax
import jax.numpy as jnp
from jax import lax
from jax.experimental import pallas as pl

_N = 20000
_TOP_K = 2048
_N_OBJS = 100
_NMS_IOU = 0.4
_IMG_H, _IMG_W = 640.0, 640.0

_LANES = 128
_ROWS = 256
_S = _ROWS * _LANES
_NEG_PAD = -3.0e38
_SUPPRESSED = -1e9


def _bitonic_sort(chans, rows, lanes):
    n = rows * lanes
    row_iota = lax.broadcasted_iota(jnp.int32, (rows, lanes), 0)
    lane_iota = lax.broadcasted_iota(jnp.int32, (rows, lanes), 1)

    def bit_clear(dist):
        if dist < lanes:
            return (lane_iota & dist) == 0
        return (row_iota & (dist // lanes)) == 0

    k = 2
    while k <= n:
        j = k // 2
        while j >= 1:
            lower = bit_clear(j)
            if j < lanes:
                part = [jnp.where(lower, jnp.roll(c, -j, axis=1),
                                  jnp.roll(c, j, axis=1)) for c in chans]
            else:
                jr = j // lanes
                part = [jnp.where(lower, jnp.roll(c, -jr, axis=0),
                                  jnp.roll(c, jr, axis=0)) for c in chans]
            cmp = ((chans[0] > part[0]) |
                   ((chans[0] == part[0]) & (chans[1] < part[1])))
            if k < n:
                keep_self = cmp == (lower == bit_clear(k))
            else:
                keep_self = cmp == lower
            chans = [jnp.where(keep_self, c, p) for c, p in zip(chans, part)]
            j //= 2
        k *= 2
    return chans


def _nms_kernel(s_ref, b0_ref, b1_ref, b2_ref, b3_ref, out_ref):
    cx = b0_ref[...] * _IMG_W
    cy = b1_ref[...] * _IMG_H
    w = b2_ref[...] * _IMG_W * 0.3 + 4.0
    h = b3_ref[...] * _IMG_H * 0.3 + 4.0
    x1 = cx - w / 2.0
    y1 = cy - h / 2.0
    x2 = cx + w / 2.0
    y2 = cy + h / 2.0

    s = s_ref[...]
    row_iota = lax.broadcasted_iota(jnp.int32, (_ROWS, _LANES), 0)
    lane_iota = lax.broadcasted_iota(jnp.int32, (_ROWS, _LANES), 1)
    idxf = (row_iota * _LANES + lane_iota).astype(jnp.float32)

    s_sorted, idx_sorted = _bitonic_sort([s, idxf], _ROWS, _LANES)
    t_rows = _TOP_K // _LANES

    dims = (((0,), (0,)), ((), ()))
    q_iota = lax.broadcasted_iota(jnp.int32, (_ROWS, 1), 0)
    l_row = lax.broadcasted_iota(jnp.int32, (1, _LANES), 1)
    planes = (x1, y1, x2, y2)
    a_cols = [[] for _ in planes]
    b_rows = [[] for _ in planes]
    for r in range(t_rows):
        idx_r = idx_sorted[r:r + 1, :].astype(jnp.int32)
        rp_row = idx_r // _LANES
        lp_row = idx_r - rp_row * _LANES
        rs_t = (q_iota == rp_row).astype(jnp.float32)
        lp_col = jnp.transpose(lp_row)
        ls = (l_row == lp_col).astype(jnp.float32)
        for c, plane in enumerate(planes):
            t_pl = lax.dot_general(rs_t, plane, dims,
                                   precision=lax.Precision.HIGHEST,
                                   preferred_element_type=jnp.float32)
            b_val = jnp.sum(t_pl * ls, axis=1, keepdims=True)
            a_cols[c].append(b_val)
            b_rows[c].append(jnp.transpose(b_val))

    ax1, ay1, ax2, ay2 = (jnp.concatenate(p, axis=0) for p in a_cols)
    bx1, by1, bx2, by2 = (jnp.concatenate(p, axis=1) for p in b_rows)
    a_area = (ax2 - ax1) * (ay2 - ay1)
    b_area = (bx2 - bx1) * (by2 - by1)

    j_col = lax.broadcasted_iota(jnp.int32, (1, _TOP_K), 1)
    chunk = 256
    max_iou = jnp.zeros((1, _TOP_K), jnp.float32)
    for c in range(_TOP_K // chunk):
        sl = slice(c * chunk, (c + 1) * chunk)
        ltx = jnp.maximum(ax1[sl], bx1)
        lty = jnp.maximum(ay1[sl], by1)
        rbx = jnp.minimum(ax2[sl], bx2)
        rby = jnp.minimum(ay2[sl], by2)
        inter = jnp.maximum(rbx - ltx, 0.0) * jnp.maximum(rby - lty, 0.0)
        iou = inter / (a_area[sl] + b_area - inter + 1e-9)
        gi = c * chunk + lax.broadcasted_iota(jnp.int32, (chunk, 1), 0)
        iou = jnp.where(gi < j_col, iou, 0.0)
        max_iou = jnp.maximum(max_iou, jnp.max(iou, axis=0, keepdims=True))

    b_sc = jnp.concatenate(
        [s_sorted[r:r + 1, :] for r in range(t_rows)], axis=1)
    kept = jnp.where(max_iou <= _NMS_IOU, b_sc, jnp.float32(_SUPPRESSED))

    def to16(v):
        return jnp.concatenate(
            [v[:, r * _LANES:(r + 1) * _LANES] for r in range(t_rows)],
            axis=0)

    pos_iota = (lax.broadcasted_iota(jnp.int32, (t_rows, _LANES), 0) *
                _LANES +
                lax.broadcasted_iota(jnp.int32, (t_rows, _LANES), 1)
                ).astype(jnp.float32)
    fch = _bitonic_sort(
        [to16(kept), pos_iota, to16(bx1), to16(by1), to16(bx2), to16(by2)],
        t_rows, _LANES)

    def to1(v):
        return jnp.concatenate(
            [v[r:r + 1, :] for r in range(t_rows)], axis=1)

    out_ref[...] = jnp.concatenate(
        [to1(fch[2]), to1(fch[3]), to1(fch[4]), to1(fch[5]), to1(fch[0]),
         jnp.zeros((3, _TOP_K), jnp.float32)], axis=0)


@jax.jit
def kernel(boxes, scores):
    pad = _S - _N
    s2d = jnp.concatenate(
        [scores, jnp.full((pad,), _NEG_PAD, jnp.float32)]).reshape(
            _ROWS, _LANES)
    cols = [jnp.concatenate(
        [boxes[:, c], jnp.zeros((pad,), jnp.float32)]).reshape(_ROWS, _LANES)
        for c in range(4)]
    buf = pl.pallas_call(
        _nms_kernel,
        out_shape=jax.ShapeDtypeStruct((8, _TOP_K), jnp.float32),
    )(s2d, *cols)
    return buf[:5, :_N_OBJS].T
